# baseline (speedup 1.0000x reference)
import functools

import jax
import jax.numpy as jnp
from jax.experimental import pallas as pl
from jax.experimental.pallas import tpu as pltpu

B = 8
N_SENT = 16
SEQ = 64
D_EM = 256
D_HID = 256
BN = B * N_SENT
TC = 8


def _gru_body(x_ref, lens_ref, wih_ref, whh_ref, bih_ref, bhh_ref,
              out_ref, h_scr):
    i = pl.program_id(0)

    @pl.when(i == 0)
    def _init():
        h_scr[...] = jnp.zeros_like(h_scr)

    lens = lens_ref[...]
    wih = wih_ref[...]
    whh = whh_ref[...]
    bih = bih_ref[...]
    bhh = bhh_ref[...]

    gis = [
        jnp.dot(x_ref[:, t, :], wih, preferred_element_type=jnp.float32)
        + bih
        for t in range(TC)
    ]

    h = h_scr[...]
    t0 = i * TC
    for t in range(TC):
        gh = jnp.dot(h, whh, preferred_element_type=jnp.float32) + bhh
        gi = gis[t]
        r = 0.5 + 0.5 * jnp.tanh(0.5 * (gi[:, :D_HID] + gh[:, :D_HID]))
        z = 0.5 + 0.5 * jnp.tanh(
            0.5 * (gi[:, D_HID:2 * D_HID] + gh[:, D_HID:2 * D_HID]))
        n = jnp.tanh(gi[:, 2 * D_HID:] + r * gh[:, 2 * D_HID:])
        h_new = n + z * (h - n)
        valid = (t0 + t) < lens
        h = jnp.where(valid, h_new, h)
    h_scr[...] = h

    @pl.when(i == pl.num_programs(0) - 1)
    def _emit():
        out_ref[...] = h


@functools.partial(jax.jit, static_argnames=())
def kernel(inputs, mask, W_ih, W_hh, b_ih, b_hh):
    x = inputs.reshape(BN, SEQ, D_EM)
    lens = mask.reshape(BN, SEQ).sum(axis=1, keepdims=True)
    bih = b_ih.reshape(1, 3 * D_HID)
    bhh = b_hh.reshape(1, 3 * D_HID)

    grid = (SEQ // TC,)
    h_final = pl.pallas_call(
        _gru_body,
        grid=grid,
        in_specs=[
            pl.BlockSpec((BN, TC, D_EM), lambda i: (0, i, 0)),
            pl.BlockSpec((BN, 1), lambda i: (0, 0)),
            pl.BlockSpec((D_EM, 3 * D_HID), lambda i: (0, 0)),
            pl.BlockSpec((D_HID, 3 * D_HID), lambda i: (0, 0)),
            pl.BlockSpec((1, 3 * D_HID), lambda i: (0, 0)),
            pl.BlockSpec((1, 3 * D_HID), lambda i: (0, 0)),
        ],
        out_specs=pl.BlockSpec((BN, D_HID), lambda i: (0, 0)),
        out_shape=jax.ShapeDtypeStruct((BN, D_HID), jnp.float32),
        scratch_shapes=[pltpu.VMEM((BN, D_HID), jnp.float32)],
    )(x, lens, W_ih, W_hh, bih, bhh)

    return h_final.reshape(B, N_SENT, D_HID)


# speedup vs baseline: 1.0332x; 1.0332x over previous
import functools

import jax
import jax.numpy as jnp
from jax.experimental import pallas as pl
from jax.experimental.pallas import tpu as pltpu

B = 8
N_SENT = 16
SEQ = 64
D_EM = 256
D_HID = 256
BN = B * N_SENT
TC = 8


def _gru_body(x_ref, lens_ref, wih_ref, whh_ref, bih_ref, bhh_ref,
              out_ref, h_scr):
    i = pl.program_id(0)

    @pl.when(i == 0)
    def _init():
        h_scr[...] = jnp.zeros_like(h_scr)

    lens = lens_ref[...]
    wih = wih_ref[...]
    whh = whh_ref[...]
    bih = bih_ref[...]
    bhh = bhh_ref[...]

    gis = [
        jnp.dot(x_ref[:, t, :], wih, preferred_element_type=jnp.float32)
        + bih
        for t in range(TC)
    ]

    t0 = i * TC

    def step(h, gi, lens_h, t):
        gh = jnp.dot(h, whh, preferred_element_type=jnp.float32) + bhh
        r = 0.5 + 0.5 * jnp.tanh(0.5 * (gi[:, :D_HID] + gh[:, :D_HID]))
        z = 0.5 + 0.5 * jnp.tanh(
            0.5 * (gi[:, D_HID:2 * D_HID] + gh[:, D_HID:2 * D_HID]))
        n = jnp.tanh(gi[:, 2 * D_HID:] + r * gh[:, 2 * D_HID:])
        h_new = n + z * (h - n)
        valid = (t0 + t) < lens_h
        return jnp.where(valid, h_new, h)

    HB = BN // 2
    ha = h_scr[:HB, :]
    hb = h_scr[HB:, :]
    la = lens[:HB, :]
    lb = lens[HB:, :]
    for t in range(TC):
        ha = step(ha, gis[t][:HB, :], la, t)
        hb = step(hb, gis[t][HB:, :], lb, t)
    h_scr[:HB, :] = ha
    h_scr[HB:, :] = hb

    @pl.when(i == pl.num_programs(0) - 1)
    def _emit():
        out_ref[:HB, :] = ha
        out_ref[HB:, :] = hb


@functools.partial(jax.jit, static_argnames=())
def kernel(inputs, mask, W_ih, W_hh, b_ih, b_hh):
    x = inputs.reshape(BN, SEQ, D_EM)
    lens = mask.reshape(BN, SEQ).sum(axis=1, keepdims=True)
    bih = b_ih.reshape(1, 3 * D_HID)
    bhh = b_hh.reshape(1, 3 * D_HID)

    grid = (SEQ // TC,)
    h_final = pl.pallas_call(
        _gru_body,
        grid=grid,
        in_specs=[
            pl.BlockSpec((BN, TC, D_EM), lambda i: (0, i, 0)),
            pl.BlockSpec((BN, 1), lambda i: (0, 0)),
            pl.BlockSpec((D_EM, 3 * D_HID), lambda i: (0, 0)),
            pl.BlockSpec((D_HID, 3 * D_HID), lambda i: (0, 0)),
            pl.BlockSpec((1, 3 * D_HID), lambda i: (0, 0)),
            pl.BlockSpec((1, 3 * D_HID), lambda i: (0, 0)),
        ],
        out_specs=pl.BlockSpec((BN, D_HID), lambda i: (0, 0)),
        out_shape=jax.ShapeDtypeStruct((BN, D_HID), jnp.float32),
        scratch_shapes=[pltpu.VMEM((BN, D_HID), jnp.float32)],
    )(x, lens, W_ih, W_hh, bih, bhh)

    return h_final.reshape(B, N_SENT, D_HID)
